# Initial kernel scaffold; baseline (speedup 1.0000x reference)
import jax
import jax.numpy as jnp
from jax.experimental import pallas as pl


def kernel(x, edge_index, W1, a_src1, a_dst1, b1, W2, a_src2, a_dst2, b2, We, be):
    raise NotImplementedError("write your pallas kernel here")


# speedup vs baseline: 32.1426x; 32.1426x over previous
import functools

import jax
import jax.numpy as jnp
from jax import lax
from jax.experimental import pallas as pl
from jax.experimental.pallas import tpu as pltpu
from jax.experimental.pallas import tpu_sc as plsc

F32 = jnp.float32
I32 = jnp.int32
HI = jax.lax.Precision.HIGHEST

NCORE = 2
NSUB = 16
NWORK = NCORE * NSUB
CHUNK = 128
NEG = -1e30


def _round_up(a, b):
    return (a + b - 1) // b * b


def _tc1_body(n, np_, x_ref, w1_ref, as_ref, ad_ref, h_out, ts_out, td_out):
    x = x_ref[...]
    h = jnp.dot(x, w1_ref[...], precision=HI)
    h_out[pl.ds(0, n), :] = h
    h_out[pl.ds(n, np_ - n), :] = jnp.zeros((np_ - n, h.shape[1]), F32)
    ts = jnp.dot(h, as_ref[...], precision=HI)
    td = jnp.dot(h, ad_ref[...], precision=HI)
    pad = jnp.full((np_ - n, 16), NEG, F32)
    ts_out[pl.ds(0, n), :] = ts
    ts_out[pl.ds(n, np_ - n), :] = pad
    td_out[pl.ds(0, n), :] = td
    td_out[pl.ds(n, np_ - n), :] = pad


def _tc2_body(n, np_, acc_ref, den_ref, b1_ref, w2_ref, as2_ref, ad2_ref,
              k16_ref, h2_out, ts_out, td_out):
    acc = acc_ref[0] + acc_ref[1]
    den = den_ref[0] + den_ref[1]
    den128 = jnp.dot(den, k16_ref[...], precision=HI)
    h1 = jnp.maximum(acc / den128 + b1_ref[...], 0.0)
    row = lax.broadcasted_iota(I32, h1.shape, 0)
    h1 = jnp.where(row < n, h1, 0.0)
    h2 = jnp.dot(h1, w2_ref[...], precision=HI)
    h2_out[...] = h2
    row16 = lax.broadcasted_iota(I32, (np_, 16), 0)
    ts = jnp.dot(h2, as2_ref[...], precision=HI)
    td = jnp.dot(h2, ad2_ref[...], precision=HI)
    ts_out[...] = jnp.where(row16 < n, ts, NEG)
    td_out[...] = jnp.where(row16 < n, td, NEG)


def _tc3_body(n, acc_ref, den_ref, b2_ref, wt_ref, wb_ref, be_ref, s_out, t_out):
    acc = acc_ref[0] + acc_ref[1]
    den = den_ref[0][:, 0:1] + den_ref[1][:, 0:1]
    z = acc / den + b2_ref[...]
    row = lax.broadcasted_iota(I32, z.shape, 0)
    z = jnp.where(row < n, z, 0.0)
    s_out[...] = jnp.dot(z, wt_ref[...], precision=HI) + be_ref[...]
    t_out[...] = jnp.dot(z, wb_ref[...], precision=HI)


def _zero_vmem(buf, rows, width):
    z = jnp.zeros((16,), F32)

    def body(r, _):
        for j in range(width // 16):
            buf[r, pl.ds(j * 16, 16)] = z
        return 0

    lax.fori_loop(0, rows, body, 0)


def _tile_row_spans(np_):
    rows_per = np_ // NSUB
    spans = []
    off = 0
    while off < rows_per:
        sz = min(CHUNK, rows_per - off)
        spans.append((off, sz))
        off += sz
    return rows_per, spans


def _sc_edge_pass(heads, ch, np_, per_w, srcp, dstp, ts, td, hp, acc_out,
                  den_out, idx_s, idx_d, rs_buf, rd_buf, hrows, msg, wbuf,
                  acc_sh, den_sh, sem0, sem1, sem2):
    cid = lax.axis_index("c")
    sid = lax.axis_index("s")
    wid = cid * NSUB + sid
    chunks = per_w // CHUNK
    rows_per, spans = _tile_row_spans(np_)
    base_r = sid * rows_per

    _zero_vmem(msg, CHUNK, ch)
    _zero_vmem(wbuf, CHUNK, 16)
    for off, sz in spans:
        pltpu.sync_copy(msg.at[pl.ds(0, sz)], acc_sh.at[pl.ds(base_r + off, sz)])
        pltpu.sync_copy(wbuf.at[pl.ds(0, sz)], den_sh.at[pl.ds(base_r + off, sz)])
    plsc.subcore_barrier()

    def chunk_body(t, _):
        base = pl.multiple_of(wid * per_w + t * CHUNK, CHUNK)
        pltpu.sync_copy(srcp.at[pl.ds(base, CHUNK)], idx_s)
        pltpu.sync_copy(dstp.at[pl.ds(base, CHUNK)], idx_d)
        cps = pltpu.async_copy(ts.at[idx_s], rs_buf, sem0)
        cpd = pltpu.async_copy(td.at[idx_d], rd_buf, sem1)
        cph = pltpu.async_copy(hp.at[idx_s], hrows, sem2)
        cps.wait()
        cpd.wait()
        cph.wait()

        def edge_body(e, _):
            al = rs_buf[e] + rd_buf[e]
            al = jnp.maximum(al, al * 0.2)
            w = jnp.exp(al)
            wbuf[e] = w
            for j in range(ch // 16):
                h = j * heads // (ch // 16)
                hv = hrows[e, pl.ds(j * 16, 16)]
                msg[e, pl.ds(j * 16, 16)] = hv * w[h]
            return 0

        lax.fori_loop(0, CHUNK, edge_body, 0)
        pltpu.sync_copy(wbuf, den_sh.at[idx_d], add=True)
        pltpu.sync_copy(msg, acc_sh.at[idx_d], add=True)
        return 0

    lax.fori_loop(0, chunks, chunk_body, 0)
    plsc.subcore_barrier()
    for off, sz in spans:
        pltpu.sync_copy(acc_sh.at[pl.ds(base_r + off, sz)],
                        acc_out.at[cid, pl.ds(base_r + off, sz)])
        pltpu.sync_copy(den_sh.at[pl.ds(base_r + off, sz)],
                        den_out.at[cid, pl.ds(base_r + off, sz)])


def _sc_edge_pred(np_, per_w, srcp, dstp, s_hbm, t_hbm, y_out, s_v, t_v,
                  idx_s, idx_d, ybuf):
    cid = lax.axis_index("c")
    sid = lax.axis_index("s")
    wid = cid * NSUB + sid
    chunks = per_w // CHUNK
    pltpu.sync_copy(s_hbm, s_v)
    pltpu.sync_copy(t_hbm, t_v)

    def chunk_body(t, _):
        base = pl.multiple_of(wid * per_w + t * CHUNK, CHUNK)
        pltpu.sync_copy(srcp.at[pl.ds(base, CHUNK)], idx_s)
        pltpu.sync_copy(dstp.at[pl.ds(base, CHUNK)], idx_d)
        for g in range(CHUNK // 16):
            ids = idx_s[pl.ds(g * 16, 16)]
            idd = idx_d[pl.ds(g * 16, 16)]
            sv = plsc.load_gather(s_v, [ids])
            tv = plsc.load_gather(t_v, [idd])
            u = sv + tv
            ybuf[pl.ds(g * 16, 16)] = 1.0 / (1.0 + jnp.exp(-u))
        pltpu.sync_copy(ybuf, y_out.at[pl.ds(base, CHUNK)])
        return 0

    lax.fori_loop(0, chunks, chunk_body, 0)


def kernel(x, edge_index, W1, a_src1, a_dst1, b1, W2, a_src2, a_dst2, b2, We, be):
    n, d = x.shape
    heads1, c1 = a_src1.shape
    c2 = a_src2.shape[1]
    dh1 = heads1 * c1
    e = edge_index.shape[1]
    np_ = _round_up(n + 1, NSUB * 8)

    loop = jnp.arange(n, dtype=edge_index.dtype)
    src = jnp.concatenate([edge_index[0], loop])
    dst = jnp.concatenate([edge_index[1], loop])
    e2 = e + n
    per_w = _round_up(e2, NWORK * CHUNK) // NWORK
    e2p = per_w * NWORK
    srcp = jnp.concatenate([src, jnp.full((e2p - e2,), n, I32)])
    dstp = jnp.concatenate([dst, jnp.full((e2p - e2,), n, I32)])
    per_w3 = _round_up(e, NWORK * CHUNK) // NWORK
    e3p = per_w3 * NWORK
    src3 = jnp.concatenate([edge_index[0], jnp.zeros((e3p - e,), I32)])
    dst3 = jnp.concatenate([edge_index[1], jnp.zeros((e3p - e,), I32)])

    sel = jnp.kron(jnp.eye(heads1, dtype=F32), jnp.ones((c1, 1), F32))
    as16 = jnp.concatenate(
        [sel * a_src1.reshape(-1, 1), jnp.zeros((dh1, 16 - heads1), F32)], axis=1)
    ad16 = jnp.concatenate(
        [sel * a_dst1.reshape(-1, 1), jnp.zeros((dh1, 16 - heads1), F32)], axis=1)
    k16 = jnp.concatenate(
        [jnp.kron(jnp.eye(heads1, dtype=F32), jnp.ones((1, c1), F32)),
         jnp.zeros((16 - heads1, dh1), F32)], axis=0)
    as2 = jnp.concatenate([a_src2.reshape(c2, 1), jnp.zeros((c2, 15), F32)], axis=1)
    ad2 = jnp.concatenate([a_dst2.reshape(c2, 1), jnp.zeros((c2, 15), F32)], axis=1)
    wtop = We[0:c2]
    wbot = We[c2:2 * c2]
    b1r = b1.reshape(1, dh1)
    b2r = b2.reshape(1, c2)
    ber = be.reshape(1, 1)

    h1p, t1s, t1d = pl.pallas_call(
        functools.partial(_tc1_body, n, np_),
        out_shape=(
            jax.ShapeDtypeStruct((np_, dh1), F32),
            jax.ShapeDtypeStruct((np_, 16), F32),
            jax.ShapeDtypeStruct((np_, 16), F32),
        ),
    )(x, W1, as16, ad16)

    mesh = plsc.VectorSubcoreMesh(core_axis_name="c", subcore_axis_name="s",
                                  num_cores=NCORE, num_subcores=NSUB)
    sc_params = pltpu.CompilerParams(use_tc_tiling_on_sc=False,
                                     needs_layout_passes=False)

    acc1, den1 = pl.kernel(
        functools.partial(_sc_edge_pass, heads1, dh1, np_, per_w),
        out_type=(
            jax.ShapeDtypeStruct((NCORE, np_, dh1), F32),
            jax.ShapeDtypeStruct((NCORE, np_, 16), F32),
        ),
        mesh=mesh,
        scratch_types=[
            pltpu.VMEM((CHUNK,), I32),
            pltpu.VMEM((CHUNK,), I32),
            pltpu.VMEM((CHUNK, 16), F32),
            pltpu.VMEM((CHUNK, 16), F32),
            pltpu.VMEM((CHUNK, dh1), F32),
            pltpu.VMEM((CHUNK, dh1), F32),
            pltpu.VMEM((CHUNK, 16), F32),
            pltpu.VMEM_SHARED((np_, dh1), F32),
            pltpu.VMEM_SHARED((np_, 16), F32),
            pltpu.SemaphoreType.DMA,
            pltpu.SemaphoreType.DMA,
            pltpu.SemaphoreType.DMA,
        ],
        compiler_params=sc_params,
    )(srcp, dstp, t1s, t1d, h1p)

    h2p, t2s, t2d = pl.pallas_call(
        functools.partial(_tc2_body, n, np_),
        out_shape=(
            jax.ShapeDtypeStruct((np_, c2), F32),
            jax.ShapeDtypeStruct((np_, 16), F32),
            jax.ShapeDtypeStruct((np_, 16), F32),
        ),
    )(acc1, den1, b1r, W2, as2, ad2, k16)

    acc2, den2 = pl.kernel(
        functools.partial(_sc_edge_pass, 1, c2, np_, per_w),
        out_type=(
            jax.ShapeDtypeStruct((NCORE, np_, c2), F32),
            jax.ShapeDtypeStruct((NCORE, np_, 16), F32),
        ),
        mesh=mesh,
        scratch_types=[
            pltpu.VMEM((CHUNK,), I32),
            pltpu.VMEM((CHUNK,), I32),
            pltpu.VMEM((CHUNK, 16), F32),
            pltpu.VMEM((CHUNK, 16), F32),
            pltpu.VMEM((CHUNK, c2), F32),
            pltpu.VMEM((CHUNK, c2), F32),
            pltpu.VMEM((CHUNK, 16), F32),
            pltpu.VMEM_SHARED((np_, c2), F32),
            pltpu.VMEM_SHARED((np_, 16), F32),
            pltpu.SemaphoreType.DMA,
            pltpu.SemaphoreType.DMA,
            pltpu.SemaphoreType.DMA,
        ],
        compiler_params=sc_params,
    )(srcp, dstp, t2s, t2d, h2p)

    s2, t2 = pl.pallas_call(
        functools.partial(_tc3_body, n),
        out_shape=(
            jax.ShapeDtypeStruct((np_, 1), F32),
            jax.ShapeDtypeStruct((np_, 1), F32),
        ),
    )(acc2, den2, b2r, wtop, wbot, ber)

    y = pl.kernel(
        functools.partial(_sc_edge_pred, np_, per_w3),
        out_type=jax.ShapeDtypeStruct((e3p,), F32),
        mesh=mesh,
        scratch_types=[
            pltpu.VMEM((np_,), F32),
            pltpu.VMEM((np_,), F32),
            pltpu.VMEM((CHUNK,), I32),
            pltpu.VMEM((CHUNK,), I32),
            pltpu.VMEM((CHUNK,), F32),
        ],
        compiler_params=sc_params,
    )(src3, dst3, s2.reshape(np_), t2.reshape(np_))

    return y[:e].reshape(e, 1)
